# speedup vs baseline: 3.1877x; 3.1877x over previous
import jax
import jax.numpy as jnp
from jax.experimental import pallas as pl

SEQ_LEN = 8192
D_MODEL = 768
BLOCK = 1024


def _copy_kernel(table_ref, out_ref):
    out_ref[...] = table_ref[...]


def kernel(x, pos_table):
    del x
    return pl.pallas_call(
        _copy_kernel,
        grid=(SEQ_LEN // BLOCK,),
        in_specs=[pl.BlockSpec((BLOCK, D_MODEL), lambda i: (i, 0))],
        out_specs=pl.BlockSpec((BLOCK, D_MODEL), lambda i: (i, 0)),
        out_shape=jax.ShapeDtypeStruct((SEQ_LEN, D_MODEL), jnp.float32),
    )(pos_table)
